# baseline (speedup 1.0000x reference)
import jax
import jax.numpy as jnp
from jax.experimental import pallas as pl

KS = 3
EPS = 1e-8


def _cconv_kernel(po_ref, piT_ref, f_ref, w_ref, b_ref, o_ref):
    pox = po_ref[:, 0:1]
    poy = po_ref[:, 1:2]
    poz = po_ref[:, 2:3]
    relx = piT_ref[0:1, :] - pox
    rely = piT_ref[1:2, :] - poy
    relz = piT_ref[2:3, :] - poz
    r2 = relx * relx + rely * rely + relz * relz
    inside = (r2 <= 1.0).astype(jnp.float32)
    rnorm = jnp.sqrt(jnp.maximum(r2, EPS))
    linf = jnp.maximum(jnp.maximum(jnp.abs(relx), jnp.abs(rely)),
                       jnp.maximum(jnp.abs(relz), EPS))
    s = rnorm / linf
    gx = jnp.clip(relx * s + 1.0, 0.0, 2.0)
    gy = jnp.clip(rely * s + 1.0, 0.0, 2.0)
    gz = jnp.clip(relz * s + 1.0, 0.0, 2.0)

    num = jnp.sum(inside, axis=1, keepdims=True)
    denom = jnp.maximum(num, 1.0)

    wx = (jnp.maximum(1.0 - gx, 0.0), 1.0 - jnp.abs(gx - 1.0),
          jnp.maximum(gx - 1.0, 0.0))
    wy = (jnp.maximum(1.0 - gy, 0.0), 1.0 - jnp.abs(gy - 1.0),
          jnp.maximum(gy - 1.0, 0.0))
    wz = tuple((jnp.maximum(1.0 - gz, 0.0) * inside).astype(jnp.bfloat16)
               if i == 0 else
               ((1.0 - jnp.abs(gz - 1.0)) * inside).astype(jnp.bfloat16)
               if i == 1 else
               (jnp.maximum(gz - 1.0, 0.0) * inside).astype(jnp.bfloat16)
               for i in range(3))

    feats = f_ref[...]
    cin = feats.shape[1]
    acc = jnp.zeros(o_ref.shape, dtype=jnp.float32)
    for vx in range(KS):
        for vy in range(KS):
            wxy = (wx[vx] * wy[vy]).astype(jnp.bfloat16)
            for vz in range(KS):
                k = (vx * KS + vy) * KS + vz
                wv = wxy * wz[vz]
                tmp = jnp.dot(wv, feats,
                              preferred_element_type=jnp.float32)
                acc = acc + jnp.dot(
                    tmp, w_ref[k * cin:(k + 1) * cin, :],
                    preferred_element_type=jnp.float32)
    o_ref[...] = jnp.maximum(acc / denom + b_ref[...], 0.0)


def kernel(features, pos_input, pos_output, extents, W, b):
    n_in, cin = features.shape
    n_out = pos_output.shape[0]
    cout = W.shape[-1]
    scale = 2.0 / extents.reshape(-1)[0]
    po = (pos_output * scale).astype(jnp.float32)
    piT = (pos_input.T * scale).astype(jnp.float32)
    wf = W.reshape(KS * KS * KS * cin, cout)
    b2 = b.reshape(1, cout)

    bo = 128
    grid = (n_out // bo,)
    out = pl.pallas_call(
        _cconv_kernel,
        grid=grid,
        in_specs=[
            pl.BlockSpec((bo, 3), lambda o: (o, 0)),
            pl.BlockSpec((3, n_in), lambda o: (0, 0)),
            pl.BlockSpec((n_in, cin), lambda o: (0, 0)),
            pl.BlockSpec((KS * KS * KS * cin, cout), lambda o: (0, 0)),
            pl.BlockSpec((1, cout), lambda o: (0, 0)),
        ],
        out_specs=pl.BlockSpec((bo, cout), lambda o: (o, 0)),
        out_shape=jax.ShapeDtypeStruct((n_out, cout), jnp.float32),
    )(po, piT, features.astype(jnp.bfloat16), wf, b2)
    return out


# speedup vs baseline: 1.1454x; 1.1454x over previous
import jax
import jax.numpy as jnp
from jax.experimental import pallas as pl
from jax.experimental.pallas import tpu as pltpu

KS = 3
EPS = 1e-8


def _cconv_kernel(po_ref, piT_ref, f_ref, whi_ref, wlo_ref, b_ref, o_ref,
                  scr_ref):
    pox = po_ref[:, 0:1]
    poy = po_ref[:, 1:2]
    poz = po_ref[:, 2:3]
    relx = piT_ref[0:1, :] - pox
    rely = piT_ref[1:2, :] - poy
    relz = piT_ref[2:3, :] - poz
    r2 = relx * relx + rely * rely + relz * relz
    inside = (r2 <= 1.0).astype(jnp.float32)
    rnorm = jnp.sqrt(jnp.maximum(r2, EPS))
    linf = jnp.maximum(jnp.maximum(jnp.abs(relx), jnp.abs(rely)),
                       jnp.maximum(jnp.abs(relz), EPS))
    s = rnorm / linf
    gx = jnp.clip(relx * s + 1.0, 0.0, 2.0)
    gy = jnp.clip(rely * s + 1.0, 0.0, 2.0)
    gz = jnp.clip(relz * s + 1.0, 0.0, 2.0)

    num = jnp.sum(inside, axis=1, keepdims=True)
    denom = jnp.maximum(num, 1.0)

    wx = (jnp.maximum(1.0 - gx, 0.0), 1.0 - jnp.abs(gx - 1.0),
          jnp.maximum(gx - 1.0, 0.0))
    wy = (jnp.maximum(1.0 - gy, 0.0), 1.0 - jnp.abs(gy - 1.0),
          jnp.maximum(gy - 1.0, 0.0))
    wz = ((jnp.maximum(1.0 - gz, 0.0) * inside).astype(jnp.bfloat16),
          ((1.0 - jnp.abs(gz - 1.0)) * inside).astype(jnp.bfloat16),
          (jnp.maximum(gz - 1.0, 0.0) * inside).astype(jnp.bfloat16))

    feats = f_ref[...]
    cin = feats.shape[1]
    for vx in range(KS):
        for vy in range(KS):
            wxy = (wx[vx] * wy[vy]).astype(jnp.bfloat16)
            for vz in range(KS):
                k = (vx * KS + vy) * KS + vz
                wv = wxy * wz[vz]
                tmp = jnp.dot(wv, feats,
                              preferred_element_type=jnp.float32)
                scr_ref[:, k * cin:(k + 1) * cin] = tmp.astype(jnp.bfloat16)

    scr = scr_ref[...]
    acc = (jnp.dot(scr, whi_ref[...], preferred_element_type=jnp.float32) +
           jnp.dot(scr, wlo_ref[...], preferred_element_type=jnp.float32))
    o_ref[...] = jnp.maximum(acc / denom + b_ref[...], 0.0)


def kernel(features, pos_input, pos_output, extents, W, b):
    n_in, cin = features.shape
    n_out = pos_output.shape[0]
    cout = W.shape[-1]
    scale = 2.0 / extents.reshape(-1)[0]
    po = (pos_output * scale).astype(jnp.float32)
    piT = (pos_input.T * scale).astype(jnp.float32)
    wf = W.reshape(KS * KS * KS * cin, cout)
    whi = wf.astype(jnp.bfloat16)
    wlo = (wf - whi.astype(jnp.float32)).astype(jnp.bfloat16)
    b2 = b.reshape(1, cout)

    bo = 128
    grid = (n_out // bo,)
    kcin = KS * KS * KS * cin
    out = pl.pallas_call(
        _cconv_kernel,
        grid=grid,
        in_specs=[
            pl.BlockSpec((bo, 3), lambda o: (o, 0)),
            pl.BlockSpec((3, n_in), lambda o: (0, 0)),
            pl.BlockSpec((n_in, cin), lambda o: (0, 0)),
            pl.BlockSpec((kcin, cout), lambda o: (0, 0)),
            pl.BlockSpec((kcin, cout), lambda o: (0, 0)),
            pl.BlockSpec((1, cout), lambda o: (0, 0)),
        ],
        out_specs=pl.BlockSpec((bo, cout), lambda o: (o, 0)),
        out_shape=jax.ShapeDtypeStruct((n_out, cout), jnp.float32),
        scratch_shapes=[pltpu.VMEM((bo, kcin), jnp.bfloat16)],
    )(po, piT, features.astype(jnp.bfloat16), whi, wlo, b2)
    return out


# speedup vs baseline: 1.3003x; 1.1352x over previous
import jax
import jax.numpy as jnp
from jax.experimental import pallas as pl
from jax.experimental.pallas import tpu as pltpu

KS = 3
EPS = 1e-8


def _cconv_kernel(po_ref, piT_ref, f_ref, whi_ref, wlo_ref, b_ref, o_ref,
                  wv_ref, scr_ref):
    bo = o_ref.shape[0]
    pox = po_ref[:, 0:1]
    poy = po_ref[:, 1:2]
    poz = po_ref[:, 2:3]
    relx = piT_ref[0:1, :] - pox
    rely = piT_ref[1:2, :] - poy
    relz = piT_ref[2:3, :] - poz
    r2 = relx * relx + rely * rely + relz * relz
    inside = (r2 <= 1.0).astype(jnp.bfloat16)
    rnorm = jnp.sqrt(jnp.maximum(r2, EPS))
    linf = jnp.maximum(jnp.maximum(jnp.abs(relx), jnp.abs(rely)),
                       jnp.maximum(jnp.abs(relz), EPS))
    s = rnorm / linf
    gx = jnp.clip(relx * s + 1.0, 0.0, 2.0).astype(jnp.bfloat16)
    gy = jnp.clip(rely * s + 1.0, 0.0, 2.0).astype(jnp.bfloat16)
    gz = jnp.clip(relz * s + 1.0, 0.0, 2.0).astype(jnp.bfloat16)

    num = jnp.sum(inside.astype(jnp.float32), axis=1, keepdims=True)
    denom = jnp.maximum(num, 1.0)

    wx = (jnp.maximum(1.0 - gx, 0.0), 1.0 - jnp.abs(gx - 1.0),
          jnp.maximum(gx - 1.0, 0.0))
    wy = (jnp.maximum(1.0 - gy, 0.0), 1.0 - jnp.abs(gy - 1.0),
          jnp.maximum(gy - 1.0, 0.0))
    wz = (jnp.maximum(1.0 - gz, 0.0) * inside,
          (1.0 - jnp.abs(gz - 1.0)) * inside,
          jnp.maximum(gz - 1.0, 0.0) * inside)

    for vx in range(KS):
        for vy in range(KS):
            wxy = wx[vx] * wy[vy]
            for vz in range(KS):
                k = (vx * KS + vy) * KS + vz
                wv_ref[k * bo:(k + 1) * bo, :] = wxy * wz[vz]

    tmpstack = jnp.dot(wv_ref[...], f_ref[...],
                       preferred_element_type=jnp.float32)
    cin = f_ref.shape[1]
    for k in range(KS * KS * KS):
        scr_ref[:, k * cin:(k + 1) * cin] = (
            tmpstack[k * bo:(k + 1) * bo, :].astype(jnp.bfloat16))

    scr = scr_ref[...]
    acc = (jnp.dot(scr, whi_ref[...], preferred_element_type=jnp.float32) +
           jnp.dot(scr, wlo_ref[...], preferred_element_type=jnp.float32))
    o_ref[...] = jnp.maximum(acc / denom + b_ref[...], 0.0)


def kernel(features, pos_input, pos_output, extents, W, b):
    n_in, cin = features.shape
    n_out = pos_output.shape[0]
    cout = W.shape[-1]
    scale = 2.0 / extents.reshape(-1)[0]
    po = (pos_output * scale).astype(jnp.float32)
    piT = (pos_input.T * scale).astype(jnp.float32)
    wf = W.reshape(KS * KS * KS * cin, cout)
    whi = wf.astype(jnp.bfloat16)
    wlo = (wf - whi.astype(jnp.float32)).astype(jnp.bfloat16)
    b2 = b.reshape(1, cout)

    bo = 128
    grid = (n_out // bo,)
    kcin = KS * KS * KS * cin
    out = pl.pallas_call(
        _cconv_kernel,
        grid=grid,
        in_specs=[
            pl.BlockSpec((bo, 3), lambda o: (o, 0)),
            pl.BlockSpec((3, n_in), lambda o: (0, 0)),
            pl.BlockSpec((n_in, cin), lambda o: (0, 0)),
            pl.BlockSpec((kcin, cout), lambda o: (0, 0)),
            pl.BlockSpec((kcin, cout), lambda o: (0, 0)),
            pl.BlockSpec((1, cout), lambda o: (0, 0)),
        ],
        out_specs=pl.BlockSpec((bo, cout), lambda o: (o, 0)),
        out_shape=jax.ShapeDtypeStruct((n_out, cout), jnp.float32),
        scratch_shapes=[pltpu.VMEM((KS * KS * KS * bo, n_in), jnp.bfloat16),
                        pltpu.VMEM((bo, kcin), jnp.bfloat16)],
    )(po, piT, features.astype(jnp.bfloat16), whi, wlo, b2)
    return out


# speedup vs baseline: 1.3425x; 1.0324x over previous
import jax
import jax.numpy as jnp
from jax.experimental import pallas as pl
from jax.experimental.pallas import tpu as pltpu

KS = 3
EPS = 1e-8


def _cconv_kernel(po_ref, piT_ref, f_ref, wf_ref, b_ref, o_ref,
                  wv_ref, scr_ref, fbf_ref, whi_ref, wlo_ref):
    bo = o_ref.shape[0]
    cin = f_ref.shape[1]

    @pl.when(pl.program_id(0) == 0)
    def _prep():
        fbf_ref[...] = f_ref[...].astype(jnp.bfloat16)
        wf = wf_ref[...]
        whi = wf.astype(jnp.bfloat16)
        whi_ref[...] = whi
        wlo_ref[...] = (wf - whi.astype(jnp.float32)).astype(jnp.bfloat16)

    pox = po_ref[:, 0:1]
    poy = po_ref[:, 1:2]
    poz = po_ref[:, 2:3]
    relx = piT_ref[0:1, :] - pox
    rely = piT_ref[1:2, :] - poy
    relz = piT_ref[2:3, :] - poz
    r2 = relx * relx + rely * rely + relz * relz
    inside = (r2 <= 1.0).astype(jnp.bfloat16)
    rnorm = jnp.sqrt(jnp.maximum(r2, EPS))
    linf = jnp.maximum(jnp.maximum(jnp.abs(relx), jnp.abs(rely)),
                       jnp.maximum(jnp.abs(relz), EPS))
    s = rnorm / linf
    gx = jnp.clip(relx * s + 1.0, 0.0, 2.0).astype(jnp.bfloat16)
    gy = jnp.clip(rely * s + 1.0, 0.0, 2.0).astype(jnp.bfloat16)
    gz = jnp.clip(relz * s + 1.0, 0.0, 2.0).astype(jnp.bfloat16)

    num = jnp.sum(inside.astype(jnp.float32), axis=1, keepdims=True)
    denom = jnp.maximum(num, 1.0)

    wx = (jnp.maximum(1.0 - gx, 0.0), 1.0 - jnp.abs(gx - 1.0),
          jnp.maximum(gx - 1.0, 0.0))
    wy = (jnp.maximum(1.0 - gy, 0.0), 1.0 - jnp.abs(gy - 1.0),
          jnp.maximum(gy - 1.0, 0.0))
    wz = (jnp.maximum(1.0 - gz, 0.0) * inside,
          (1.0 - jnp.abs(gz - 1.0)) * inside,
          jnp.maximum(gz - 1.0, 0.0) * inside)

    for vx in range(KS):
        for vy in range(KS):
            wxy = wx[vx] * wy[vy]
            for vz in range(KS):
                k = (vx * KS + vy) * KS + vz
                wv_ref[k * bo:(k + 1) * bo, :] = wxy * wz[vz]

    tmpstack = jnp.dot(wv_ref[...], fbf_ref[...],
                       preferred_element_type=jnp.float32)
    for k in range(KS * KS * KS):
        scr_ref[:, k * cin:(k + 1) * cin] = (
            tmpstack[k * bo:(k + 1) * bo, :].astype(jnp.bfloat16))

    scr = scr_ref[...]
    acc = (jnp.dot(scr, whi_ref[...], preferred_element_type=jnp.float32) +
           jnp.dot(scr, wlo_ref[...], preferred_element_type=jnp.float32))
    o_ref[...] = jnp.maximum(acc / denom + b_ref[...], 0.0)


def kernel(features, pos_input, pos_output, extents, W, b):
    n_in, cin = features.shape
    n_out = pos_output.shape[0]
    cout = W.shape[-1]
    scale = 2.0 / extents.reshape(-1)[0]
    po = (pos_output * scale).astype(jnp.float32)
    piT = (pos_input.T * scale).astype(jnp.float32)
    wf = W.reshape(KS * KS * KS * cin, cout)
    b2 = b.reshape(1, cout)

    bo = 128
    grid = (n_out // bo,)
    kcin = KS * KS * KS * cin
    out = pl.pallas_call(
        _cconv_kernel,
        grid=grid,
        in_specs=[
            pl.BlockSpec((bo, 3), lambda o: (o, 0)),
            pl.BlockSpec((3, n_in), lambda o: (0, 0)),
            pl.BlockSpec((n_in, cin), lambda o: (0, 0)),
            pl.BlockSpec((kcin, cout), lambda o: (0, 0)),
            pl.BlockSpec((1, cout), lambda o: (0, 0)),
        ],
        out_specs=pl.BlockSpec((bo, cout), lambda o: (o, 0)),
        out_shape=jax.ShapeDtypeStruct((n_out, cout), jnp.float32),
        scratch_shapes=[pltpu.VMEM((KS * KS * KS * bo, n_in), jnp.bfloat16),
                        pltpu.VMEM((bo, kcin), jnp.bfloat16),
                        pltpu.VMEM((n_in, cin), jnp.bfloat16),
                        pltpu.VMEM((kcin, cout), jnp.bfloat16),
                        pltpu.VMEM((kcin, cout), jnp.bfloat16)],
    )(po, piT, features, wf, b2)
    return out
